# baseline (speedup 1.0000x reference)
import functools

import jax
import jax.numpy as jnp
from jax import lax
from jax.experimental import pallas as pl
from jax.experimental.pallas import tpu as pltpu
from jax.experimental.pallas import tpu_sc as plsc

_L = 16


def _make_sc_call(n_batch, n_time, row_w, ncs, horizon, n_workers, chunk_elems):
    table_size = horizon * ncs
    assert n_batch % (n_workers * chunk_elems) == 0
    elems_per_w = n_batch // n_workers
    n_chunks = elems_per_w // chunk_elems
    assert n_chunks % 2 == 0
    chunk_rows = chunk_elems * n_time
    assert chunk_rows % _L == 0 and chunk_rows % 8 == 0
    groups = chunk_rows // _L
    n_rows = n_batch * n_time

    mesh = plsc.VectorSubcoreMesh(core_axis_name="c", subcore_axis_name="s")

    @functools.partial(
        pl.kernel,
        mesh=mesh,
        compiler_params=pltpu.CompilerParams(
            needs_layout_passes=False,
            use_tc_tiling_on_sc=False,
        ),
        out_type=jax.ShapeDtypeStruct((n_rows,), jnp.int32),
        scratch_types=[
            pltpu.VMEM((chunk_elems, n_time, row_w), jnp.float32),
            pltpu.VMEM((chunk_elems, n_time, row_w), jnp.float32),
            pltpu.VMEM((table_size,), jnp.int32),
            pltpu.VMEM((chunk_rows,), jnp.int32),
            pltpu.SemaphoreType.DMA,
            pltpu.SemaphoreType.DMA,
        ],
    )
    def sc_call(obs_hbm, dp_hbm, out_hbm, buf0, buf1, dpv, outv, sem0, sem1):
        num_cores = 2
        wid = lax.axis_index("s") * num_cores + lax.axis_index("c")
        base_elem = wid * elems_per_w
        base_row = base_elem * n_time

        pltpu.sync_copy(dp_hbm, dpv)

        def fetch(buf, sem, ch):
            pltpu.make_async_copy(
                obs_hbm.at[pl.ds(base_elem + ch * chunk_elems, chunk_elems)],
                buf,
                sem,
            ).start()

        def wait(buf, sem):
            pltpu.make_async_copy(
                obs_hbm.at[pl.ds(base_elem, chunk_elems)], buf, sem
            ).wait()

        iota = lax.iota(jnp.int32, _L)

        def compute_chunk(buf, ch):
            def g_body(g, carry):
                r = g * _L + iota
                recip = (1 << 18) // n_time + 1
                e = (r * recip) >> 18
                t = r - e * n_time
                cvec = jnp.zeros((_L,), jnp.int32)
                m = plsc.load_gather(buf, [e, t, cvec])
                s = jnp.zeros((_L,), jnp.int32)
                for c in range(1, ncs):
                    cvec = cvec + 1
                    v = plsc.load_gather(buf, [e, t, cvec])
                    gt = v > m
                    m = jnp.where(gt, v, m)
                    s = jnp.where(gt, c, s)
                tau = plsc.load_gather(buf, [e, t, cvec + 1])
                ti = (tau * float(horizon)).astype(jnp.int32)
                ti = jnp.clip(ti, 0, horizon - 1)
                act = plsc.load_gather(dpv, [ti * ncs + s])
                outv[pl.ds(g * _L, _L)] = act
                return carry

            lax.fori_loop(0, groups, g_body, 0, unroll=False)
            pltpu.sync_copy(
                outv, out_hbm.at[pl.ds(base_row + ch * chunk_rows, chunk_rows)]
            )

        fetch(buf0, sem0, 0)
        fetch(buf1, sem1, 1)

        def loop_body(i, carry):
            ch0 = 2 * i
            wait(buf0, sem0)
            compute_chunk(buf0, ch0)
            fetch(buf0, sem0, jnp.minimum(ch0 + 2, n_chunks - 1))
            ch1 = ch0 + 1
            wait(buf1, sem1)
            compute_chunk(buf1, ch1)
            fetch(buf1, sem1, jnp.minimum(ch1 + 2, n_chunks - 1))
            return carry

        lax.fori_loop(0, n_chunks // 2, loop_body, 0, unroll=False)
        wait(buf0, sem0)
        wait(buf1, sem1)

    return sc_call


def kernel(observation, dp_table):
    b, t, cw = observation.shape
    horizon, ncs = dp_table.shape
    info = plsc.get_sparse_core_info()
    n_workers = info.num_cores * info.num_subcores

    sc_call = _make_sc_call(
        n_batch=b,
        n_time=t,
        row_w=cw,
        ncs=ncs,
        horizon=horizon,
        n_workers=n_workers,
        chunk_elems=8,
    )
    dp_flat = dp_table.reshape(-1)
    out = sc_call(observation, dp_flat)
    return out.reshape(b, t)
